# baseline (speedup 1.0000x reference)
import functools

import jax
import jax.numpy as jnp
from jax import lax
from jax.experimental import pallas as pl
from jax.experimental.pallas import tpu as pltpu
from jax.experimental.pallas import tpu_sc as plsc

NUM_EXPERTS = 8
TOP_K = 2
HIDDEN = 2048
D_MODEL = 1024
AUX_W = 0.01
EPAD = 128
TILE = 128
N_TOK = 4096
N_GRID = (N_TOK * TOP_K) // TILE + NUM_EXPERTS
NPAD = N_GRID * TILE
CHUNK = 128

_sc_kernels_cache = {}


def _router_body(x_ref, w_ref, b_ref,
                 probs_ref, sel_ref, misc_ref, tmap_ref, w0b_ref, w1b_ref,
                 neg_s):
    n = x_ref.shape[0]
    nch = n // CHUNK
    x = x_ref[...]
    W = w_ref[...]
    logits = jnp.dot(x, W, preferred_element_type=jnp.float32) + b_ref[...]
    colf = lax.broadcasted_iota(jnp.int32, (n, EPAD), 1).astype(jnp.float32)
    neg_s[...] = jnp.where(colf < NUM_EXPERTS, logits, -1e30)

    cc = lax.broadcasted_iota(jnp.int32, (CHUNK, EPAD), 1).astype(jnp.float32)
    rr = lax.broadcasted_iota(jnp.int32, (CHUNK, EPAD), 0).astype(jnp.float32)
    ltri = (cc < rr).astype(jnp.float32)
    utri = (rr < cc).astype(jnp.float32)

    def phase1(c, carry):
        off0, off1, psum = carry
        neg = neg_s[pl.ds(c * CHUNK, CHUNK), :]
        m0 = jnp.max(neg, axis=1, keepdims=True)
        i0 = jnp.min(jnp.where(neg == m0, cc, 1e9), axis=1, keepdims=True)
        mask0 = (cc == i0).astype(jnp.float32)
        neg1 = jnp.where(cc == i0, -1e30, neg)
        m1 = jnp.max(neg1, axis=1, keepdims=True)
        i1 = jnp.min(jnp.where(neg1 == m1, cc, 1e9), axis=1, keepdims=True)
        mask1 = (cc == i1).astype(jnp.float32)
        ex = jnp.exp(neg - m0)
        probs = ex / jnp.sum(ex, axis=1, keepdims=True)
        probs_ref[pl.ds(c * CHUNK, CHUNK), :] = probs
        t = jnp.exp(m1 - m0)
        w0 = 1.0 / (1.0 + t)
        w1 = t / (1.0 + t)
        w0b_ref[pl.ds(c * CHUNK, CHUNK), :] = jnp.broadcast_to(w0, (CHUNK, EPAD))
        w1b_ref[pl.ds(c * CHUNK, CHUNK), :] = jnp.broadcast_to(w1, (CHUNK, EPAD))
        cum0 = jnp.dot(ltri, mask0, preferred_element_type=jnp.float32)
        cum1 = jnp.dot(ltri, mask1, preferred_element_type=jnp.float32)
        rank0 = (jnp.sum(cum0 * mask0, axis=1, keepdims=True)
                 + jnp.sum(off0 * mask0, axis=1, keepdims=True))
        rank1 = (jnp.sum(cum1 * mask1, axis=1, keepdims=True)
                 + jnp.sum(off1 * mask1, axis=1, keepdims=True))
        sel = jnp.where(cc == 0, w0, 0.0)
        sel = sel + jnp.where(cc == 1, w1, 0.0)
        sel = sel + jnp.where(cc == 2, i0, 0.0)
        sel = sel + jnp.where(cc == 3, i1, 0.0)
        sel = sel + jnp.where(cc == 4, rank0, 0.0)
        sel = sel + jnp.where(cc == 5, rank1, 0.0)
        sel_ref[pl.ds(c * CHUNK, CHUNK), :] = sel
        return (off0 + jnp.sum(mask0, axis=0, keepdims=True),
                off1 + jnp.sum(mask1, axis=0, keepdims=True),
                psum + jnp.sum(probs, axis=0, keepdims=True))

    zero = jnp.zeros((1, EPAD), jnp.float32)
    cnt0, cnt1, psum = lax.fori_loop(0, nch, phase1, (zero, zero, zero))

    cnt = cnt0 + cnt1
    pm = psum / n
    aux = jnp.sum(cnt * pm) * (NUM_EXPERTS * AUX_W / (n * TOP_K))
    pc = jnp.floor((cnt + (TILE - 1)) / TILE) * TILE
    padbase = jnp.dot(pc, utri, preferred_element_type=jnp.float32)

    row8 = lax.broadcasted_iota(jnp.int32, (8, EPAD), 0)
    misc = jnp.where(row8 == 0, jnp.broadcast_to(cnt, (8, EPAD)), 0.0)
    misc = misc + jnp.where(row8 == 1, jnp.broadcast_to(pm, (8, EPAD)), 0.0)
    misc = misc + jnp.where(row8 == 2, aux, 0.0)
    misc_ref[...] = misc

    pbb = jnp.broadcast_to(padbase, (CHUNK, EPAD))
    c0b = jnp.broadcast_to(cnt0, (CHUNK, EPAD))

    def phase2(c, _):
        sel = sel_ref[pl.ds(c * CHUNK, CHUNK), :]
        i0 = jnp.sum(jnp.where(cc == 2, sel, 0.0), axis=1, keepdims=True)
        i1 = jnp.sum(jnp.where(cc == 3, sel, 0.0), axis=1, keepdims=True)
        rank0 = jnp.sum(jnp.where(cc == 4, sel, 0.0), axis=1, keepdims=True)
        rank1 = jnp.sum(jnp.where(cc == 5, sel, 0.0), axis=1, keepdims=True)
        pb0 = jnp.sum(jnp.where(cc == i0, pbb, 0.0), axis=1, keepdims=True)
        pb1 = jnp.sum(jnp.where(cc == i1, pbb, 0.0), axis=1, keepdims=True)
        c01 = jnp.sum(jnp.where(cc == i1, c0b, 0.0), axis=1, keepdims=True)
        pos0 = pb0 + rank0
        pos1 = pb1 + c01 + rank1
        sel2 = (jnp.where(cc == 4, pos0 - rank0, 0.0)
                + jnp.where(cc == 5, pos1 - rank1, 0.0))
        sel_ref[pl.ds(c * CHUNK, CHUNK), :] = sel + sel2
        return 0

    lax.fori_loop(0, nch, phase2, 0)

    gi = lax.broadcasted_iota(jnp.int32, (CHUNK, EPAD), 0).astype(jnp.float32)
    ce = lax.broadcasted_iota(jnp.int32, (CHUNK, EPAD), 1).astype(jnp.float32)
    ende = jnp.broadcast_to(padbase + pc, (CHUNK, EPAD))
    ind = ((gi * TILE >= ende) & (ce < NUM_EXPERTS)).astype(jnp.int32)
    te = jnp.sum(ind, axis=1, keepdims=True)
    te = jnp.minimum(te, NUM_EXPERTS - 1)
    tmap_ref[...] = jnp.broadcast_to(te, (CHUNK, EPAD))


def _dispatch_body(x_hbm, p0_hbm, p1_hbm, xs_hbm, xb, idx0, idx1, s0, s1):
    wid = lax.axis_index("s") * 2 + lax.axis_index("c")
    base = wid * 128
    for sub in range(2):
        tok = pl.multiple_of(base + sub * 64, 64)
        pltpu.sync_copy(p0_hbm.at[pl.ds(tok, 64)], idx0)
        pltpu.sync_copy(p1_hbm.at[pl.ds(tok, 64)], idx1)
        pltpu.sync_copy(x_hbm.at[pl.ds(tok, 64)], xb)
        c0 = pltpu.async_copy(xb, xs_hbm.at[idx0], s0)
        c1 = pltpu.async_copy(xb, xs_hbm.at[idx1], s1)
        c0.wait()
        c1.wait()


def _gmm_body(te_ref, xs_ref, upw_ref, upb_ref, dnw_ref, dnb_ref, out_ref):
    x = xs_ref[...].astype(jnp.bfloat16)
    h = jnp.dot(x, upw_ref[0], preferred_element_type=jnp.float32)
    h = jax.nn.gelu(h + upb_ref[0, 0][None, :])
    y = jnp.dot(h.astype(jnp.bfloat16), dnw_ref[0],
                preferred_element_type=jnp.float32)
    out_ref[...] = y + dnb_ref[0, 0][None, :]


def _combine_body(ys_hbm, p0_hbm, p1_hbm, w0b_hbm, w1b_hbm, out_hbm,
                  y0, y1, ob, idx0, idx1, w0v, w1v, s0, s1):
    wid = lax.axis_index("s") * 2 + lax.axis_index("c")
    base = wid * 128

    def chunk(ch, _):
        tok = pl.multiple_of(base + ch * 16, 16)
        pltpu.sync_copy(p0_hbm.at[pl.ds(tok, 16)], idx0)
        pltpu.sync_copy(p1_hbm.at[pl.ds(tok, 16)], idx1)
        c0 = pltpu.async_copy(ys_hbm.at[idx0], y0, s0)
        c1 = pltpu.async_copy(ys_hbm.at[idx1], y1, s1)
        pltpu.sync_copy(w0b_hbm.at[pl.ds(tok, 16)], w0v)
        pltpu.sync_copy(w1b_hbm.at[pl.ds(tok, 16)], w1v)
        c0.wait()
        c1.wait()

        def row(r, _):
            w0s = w0v[r, pl.ds(0, 16)]
            w1s = w1v[r, pl.ds(0, 16)]
            for v in range(D_MODEL // 16):
                ob[r, pl.ds(v * 16, 16)] = (
                    w0s * y0[r, pl.ds(v * 16, 16)]
                    + w1s * y1[r, pl.ds(v * 16, 16)])
            return 0

        lax.fori_loop(0, 16, row, 0)
        pltpu.sync_copy(ob, out_hbm.at[pl.ds(tok, 16)])
        return 0

    lax.fori_loop(0, 8, chunk, 0)


def _get_sc_kernels():
    if "k" not in _sc_kernels_cache:
        mesh = plsc.VectorSubcoreMesh(core_axis_name="c", subcore_axis_name="s")
        dispatch = pl.kernel(
            _dispatch_body, mesh=mesh,
            out_type=jax.ShapeDtypeStruct((NPAD, D_MODEL), jnp.float32),
            scratch_types=[
                pltpu.VMEM((64, D_MODEL), jnp.float32),
                pltpu.VMEM((64,), jnp.int32),
                pltpu.VMEM((64,), jnp.int32),
                pltpu.SemaphoreType.DMA,
                pltpu.SemaphoreType.DMA,
            ])
        combine = pl.kernel(
            _combine_body, mesh=mesh,
            out_type=jax.ShapeDtypeStruct((N_TOK, D_MODEL), jnp.float32),
            scratch_types=[
                pltpu.VMEM((16, D_MODEL), jnp.float32),
                pltpu.VMEM((16, D_MODEL), jnp.float32),
                pltpu.VMEM((16, D_MODEL), jnp.float32),
                pltpu.VMEM((16,), jnp.int32),
                pltpu.VMEM((16,), jnp.int32),
                pltpu.VMEM((16, EPAD), jnp.float32),
                pltpu.VMEM((16, EPAD), jnp.float32),
                pltpu.SemaphoreType.DMA,
                pltpu.SemaphoreType.DMA,
            ])
        _sc_kernels_cache["k"] = (dispatch, combine)
    return _sc_kernels_cache["k"]


def kernel(x, router_W, router_b, up_W, up_b, down_W, down_b):
    _dispatch, _combine = _get_sc_kernels()
    B, S, D = x.shape
    N = B * S
    x2 = x.reshape(N, D)
    Wp = jnp.zeros((D, EPAD), jnp.float32).at[:, :NUM_EXPERTS].set(router_W)
    bp = jnp.zeros((1, EPAD), jnp.float32).at[0, :NUM_EXPERTS].set(router_b)

    probs_p, sel, misc, tmap, w0b, w1b = pl.pallas_call(
        _router_body,
        out_shape=[
            jax.ShapeDtypeStruct((N, EPAD), jnp.float32),
            jax.ShapeDtypeStruct((N, EPAD), jnp.float32),
            jax.ShapeDtypeStruct((8, EPAD), jnp.float32),
            jax.ShapeDtypeStruct((CHUNK, EPAD), jnp.int32),
            jax.ShapeDtypeStruct((N, EPAD), jnp.float32),
            jax.ShapeDtypeStruct((N, EPAD), jnp.float32),
        ],
        scratch_shapes=[pltpu.VMEM((N, EPAD), jnp.float32)],
        compiler_params=pltpu.CompilerParams(
            vmem_limit_bytes=63 * 1024 * 1024),
    )(x2, Wp, bp)

    pos0 = sel[:, 4].astype(jnp.int32)
    pos1 = sel[:, 5].astype(jnp.int32)
    te = tmap[:N_GRID, 0]

    xs = _dispatch(x2, pos0, pos1)

    ys = pl.pallas_call(
        _gmm_body,
        grid_spec=pltpu.PrefetchScalarGridSpec(
            num_scalar_prefetch=1,
            grid=(N_GRID,),
            in_specs=[
                pl.BlockSpec((TILE, D), lambda g, te: (g, 0)),
                pl.BlockSpec((1, D, HIDDEN), lambda g, te: (te[g], 0, 0)),
                pl.BlockSpec((1, 1, HIDDEN), lambda g, te: (te[g], 0, 0)),
                pl.BlockSpec((1, HIDDEN, D), lambda g, te: (te[g], 0, 0)),
                pl.BlockSpec((1, 1, D), lambda g, te: (te[g], 0, 0)),
            ],
            out_specs=pl.BlockSpec((TILE, D), lambda g, te: (g, 0)),
        ),
        out_shape=jax.ShapeDtypeStruct((NPAD, D), jnp.float32),
        compiler_params=pltpu.CompilerParams(
            dimension_semantics=("arbitrary",),
            vmem_limit_bytes=63 * 1024 * 1024),
    )(te, xs, up_W.astype(jnp.bfloat16), up_b.reshape(NUM_EXPERTS, 1, HIDDEN),
      down_W.astype(jnp.bfloat16), down_b.reshape(NUM_EXPERTS, 1, D))

    out2 = _combine(ys, pos0, pos1, w0b, w1b)

    router_probs = probs_p[:, :NUM_EXPERTS].reshape(B, S, NUM_EXPERTS)
    aux_loss = misc[2, 0]
    return (out2.reshape(B, S, D), aux_loss, router_probs)


# speedup vs baseline: 1.2207x; 1.2207x over previous
import functools

import jax
import jax.numpy as jnp
from jax import lax
from jax.experimental import pallas as pl
from jax.experimental.pallas import tpu as pltpu
from jax.experimental.pallas import tpu_sc as plsc

NUM_EXPERTS = 8
TOP_K = 2
HIDDEN = 2048
D_MODEL = 1024
AUX_W = 0.01
EPAD = 128
TILE = 256
N_TOK = 4096
N_GRID = (N_TOK * TOP_K) // TILE + NUM_EXPERTS
NPAD = N_GRID * TILE
CHUNK = 128

_sc_kernels_cache = {}


def _router_body(x_ref, w_ref, b_ref,
                 probs_ref, sel_ref, misc_ref, tmap_ref, w0b_ref, w1b_ref,
                 neg_s):
    n = x_ref.shape[0]
    nch = n // CHUNK
    x = x_ref[...]
    W = w_ref[...]
    logits = jnp.dot(x, W, preferred_element_type=jnp.float32) + b_ref[...]
    colf = lax.broadcasted_iota(jnp.int32, (n, EPAD), 1).astype(jnp.float32)
    neg_s[...] = jnp.where(colf < NUM_EXPERTS, logits, -1e30)

    cc = lax.broadcasted_iota(jnp.int32, (CHUNK, EPAD), 1).astype(jnp.float32)
    rr = lax.broadcasted_iota(jnp.int32, (CHUNK, EPAD), 0).astype(jnp.float32)
    ltri = (cc < rr).astype(jnp.float32)
    utri = (rr < cc).astype(jnp.float32)

    def phase1(c, carry):
        off0, off1, psum = carry
        neg = neg_s[pl.ds(c * CHUNK, CHUNK), :]
        m0 = jnp.max(neg, axis=1, keepdims=True)
        i0 = jnp.min(jnp.where(neg == m0, cc, 1e9), axis=1, keepdims=True)
        mask0 = (cc == i0).astype(jnp.float32)
        neg1 = jnp.where(cc == i0, -1e30, neg)
        m1 = jnp.max(neg1, axis=1, keepdims=True)
        i1 = jnp.min(jnp.where(neg1 == m1, cc, 1e9), axis=1, keepdims=True)
        mask1 = (cc == i1).astype(jnp.float32)
        ex = jnp.exp(neg - m0)
        probs = ex / jnp.sum(ex, axis=1, keepdims=True)
        probs_ref[pl.ds(c * CHUNK, CHUNK), :] = probs
        t = jnp.exp(m1 - m0)
        w0 = 1.0 / (1.0 + t)
        w1 = t / (1.0 + t)
        w0b_ref[pl.ds(c * CHUNK, CHUNK), :] = jnp.broadcast_to(w0, (CHUNK, EPAD))
        w1b_ref[pl.ds(c * CHUNK, CHUNK), :] = jnp.broadcast_to(w1, (CHUNK, EPAD))
        cum0 = jnp.dot(ltri, mask0, preferred_element_type=jnp.float32)
        cum1 = jnp.dot(ltri, mask1, preferred_element_type=jnp.float32)
        rank0 = (jnp.sum(cum0 * mask0, axis=1, keepdims=True)
                 + jnp.sum(off0 * mask0, axis=1, keepdims=True))
        rank1 = (jnp.sum(cum1 * mask1, axis=1, keepdims=True)
                 + jnp.sum(off1 * mask1, axis=1, keepdims=True))
        sel = jnp.where(cc == 0, w0, 0.0)
        sel = sel + jnp.where(cc == 1, w1, 0.0)
        sel = sel + jnp.where(cc == 2, i0, 0.0)
        sel = sel + jnp.where(cc == 3, i1, 0.0)
        sel = sel + jnp.where(cc == 4, rank0, 0.0)
        sel = sel + jnp.where(cc == 5, rank1, 0.0)
        sel_ref[pl.ds(c * CHUNK, CHUNK), :] = sel
        return (off0 + jnp.sum(mask0, axis=0, keepdims=True),
                off1 + jnp.sum(mask1, axis=0, keepdims=True),
                psum + jnp.sum(probs, axis=0, keepdims=True))

    zero = jnp.zeros((1, EPAD), jnp.float32)
    cnt0, cnt1, psum = lax.fori_loop(0, nch, phase1, (zero, zero, zero))

    cnt = cnt0 + cnt1
    pm = psum / n
    aux = jnp.sum(cnt * pm) * (NUM_EXPERTS * AUX_W / (n * TOP_K))
    pc = jnp.floor((cnt + (TILE - 1)) / TILE) * TILE
    padbase = jnp.dot(pc, utri, preferred_element_type=jnp.float32)

    row8 = lax.broadcasted_iota(jnp.int32, (8, EPAD), 0)
    misc = jnp.where(row8 == 0, jnp.broadcast_to(cnt, (8, EPAD)), 0.0)
    misc = misc + jnp.where(row8 == 1, jnp.broadcast_to(pm, (8, EPAD)), 0.0)
    misc = misc + jnp.where(row8 == 2, aux, 0.0)
    misc_ref[...] = misc

    pbb = jnp.broadcast_to(padbase, (CHUNK, EPAD))
    c0b = jnp.broadcast_to(cnt0, (CHUNK, EPAD))

    def phase2(c, _):
        sel = sel_ref[pl.ds(c * CHUNK, CHUNK), :]
        i0 = jnp.sum(jnp.where(cc == 2, sel, 0.0), axis=1, keepdims=True)
        i1 = jnp.sum(jnp.where(cc == 3, sel, 0.0), axis=1, keepdims=True)
        rank0 = jnp.sum(jnp.where(cc == 4, sel, 0.0), axis=1, keepdims=True)
        rank1 = jnp.sum(jnp.where(cc == 5, sel, 0.0), axis=1, keepdims=True)
        pb0 = jnp.sum(jnp.where(cc == i0, pbb, 0.0), axis=1, keepdims=True)
        pb1 = jnp.sum(jnp.where(cc == i1, pbb, 0.0), axis=1, keepdims=True)
        c01 = jnp.sum(jnp.where(cc == i1, c0b, 0.0), axis=1, keepdims=True)
        pos0 = pb0 + rank0
        pos1 = pb1 + c01 + rank1
        sel2 = (jnp.where(cc == 4, pos0 - rank0, 0.0)
                + jnp.where(cc == 5, pos1 - rank1, 0.0))
        sel_ref[pl.ds(c * CHUNK, CHUNK), :] = sel + sel2
        return 0

    lax.fori_loop(0, nch, phase2, 0)

    gi = lax.broadcasted_iota(jnp.int32, (CHUNK, EPAD), 0).astype(jnp.float32)
    ce = lax.broadcasted_iota(jnp.int32, (CHUNK, EPAD), 1).astype(jnp.float32)
    ende = jnp.broadcast_to(padbase + pc, (CHUNK, EPAD))
    ind = ((gi * TILE >= ende) & (ce < NUM_EXPERTS)).astype(jnp.int32)
    te = jnp.sum(ind, axis=1, keepdims=True)
    te = jnp.minimum(te, NUM_EXPERTS - 1)
    totpad = jnp.sum(pc)
    gcol = lax.broadcasted_iota(jnp.int32, (CHUNK, 1), 0).astype(jnp.float32)
    te = jnp.where(gcol * TILE >= totpad, -1, te)
    tmap_ref[...] = jnp.broadcast_to(te, (CHUNK, EPAD))


def _dispatch_body(x_hbm, p0_hbm, p1_hbm, xs_hbm, xb, idx0, idx1, s0, s1):
    wid = lax.axis_index("s") * 2 + lax.axis_index("c")
    base = wid * 128
    for sub in range(2):
        tok = pl.multiple_of(base + sub * 64, 64)
        pltpu.sync_copy(p0_hbm.at[pl.ds(tok, 64)], idx0)
        pltpu.sync_copy(p1_hbm.at[pl.ds(tok, 64)], idx1)
        pltpu.sync_copy(x_hbm.at[pl.ds(tok, 64)], xb)
        c0 = pltpu.async_copy(xb, xs_hbm.at[idx0], s0)
        c1 = pltpu.async_copy(xb, xs_hbm.at[idx1], s1)
        c0.wait()
        c1.wait()


def _gmm_body(te_ref, xs_ref, upw_ref, upb_ref, dnw_ref, dnb_ref, out_ref):
    g = pl.program_id(0)

    @pl.when(te_ref[g] >= 0)
    def _():
        x = xs_ref[...]
        h = jnp.dot(x, upw_ref[0], preferred_element_type=jnp.float32)
        h = jax.nn.gelu(h + upb_ref[0, 0][None, :])
        y = jnp.dot(h, dnw_ref[0], preferred_element_type=jnp.float32)
        out_ref[...] = y + dnb_ref[0, 0][None, :]


def _combine_body(ys_hbm, p0_hbm, p1_hbm, w0b_hbm, w1b_hbm, out_hbm,
                  y0, y1, ob, idx0, idx1, w0v, w1v, s0, s1):
    wid = lax.axis_index("s") * 2 + lax.axis_index("c")
    base = wid * 128

    def chunk(ch, _):
        tok = pl.multiple_of(base + ch * 16, 16)
        pltpu.sync_copy(p0_hbm.at[pl.ds(tok, 16)], idx0)
        pltpu.sync_copy(p1_hbm.at[pl.ds(tok, 16)], idx1)
        c0 = pltpu.async_copy(ys_hbm.at[idx0], y0, s0)
        c1 = pltpu.async_copy(ys_hbm.at[idx1], y1, s1)
        pltpu.sync_copy(w0b_hbm.at[pl.ds(tok, 16)], w0v)
        pltpu.sync_copy(w1b_hbm.at[pl.ds(tok, 16)], w1v)
        c0.wait()
        c1.wait()

        def row(r, _):
            w0s = w0v[r, pl.ds(0, 16)]
            w1s = w1v[r, pl.ds(0, 16)]
            for v in range(D_MODEL // 16):
                ob[r, pl.ds(v * 16, 16)] = (
                    w0s * y0[r, pl.ds(v * 16, 16)]
                    + w1s * y1[r, pl.ds(v * 16, 16)])
            return 0

        lax.fori_loop(0, 16, row, 0)
        pltpu.sync_copy(ob, out_hbm.at[pl.ds(tok, 16)])
        return 0

    lax.fori_loop(0, 8, chunk, 0)


def _get_sc_kernels():
    if "k" not in _sc_kernels_cache:
        mesh = plsc.VectorSubcoreMesh(core_axis_name="c", subcore_axis_name="s")
        dispatch = pl.kernel(
            _dispatch_body, mesh=mesh,
            out_type=jax.ShapeDtypeStruct((NPAD, D_MODEL), jnp.float32),
            scratch_types=[
                pltpu.VMEM((64, D_MODEL), jnp.float32),
                pltpu.VMEM((64,), jnp.int32),
                pltpu.VMEM((64,), jnp.int32),
                pltpu.SemaphoreType.DMA,
                pltpu.SemaphoreType.DMA,
            ])
        combine = pl.kernel(
            _combine_body, mesh=mesh,
            out_type=jax.ShapeDtypeStruct((N_TOK, D_MODEL), jnp.float32),
            scratch_types=[
                pltpu.VMEM((16, D_MODEL), jnp.float32),
                pltpu.VMEM((16, D_MODEL), jnp.float32),
                pltpu.VMEM((16, D_MODEL), jnp.float32),
                pltpu.VMEM((16,), jnp.int32),
                pltpu.VMEM((16,), jnp.int32),
                pltpu.VMEM((16, EPAD), jnp.float32),
                pltpu.VMEM((16, EPAD), jnp.float32),
                pltpu.SemaphoreType.DMA,
                pltpu.SemaphoreType.DMA,
            ])
        _sc_kernels_cache["k"] = (dispatch, combine)
    return _sc_kernels_cache["k"]


def kernel(x, router_W, router_b, up_W, up_b, down_W, down_b):
    _dispatch, _combine = _get_sc_kernels()
    B, S, D = x.shape
    N = B * S
    x2 = x.reshape(N, D)
    Wp = jnp.zeros((D, EPAD), jnp.float32).at[:, :NUM_EXPERTS].set(router_W)
    bp = jnp.zeros((1, EPAD), jnp.float32).at[0, :NUM_EXPERTS].set(router_b)

    probs_p, sel, misc, tmap, w0b, w1b = pl.pallas_call(
        _router_body,
        out_shape=[
            jax.ShapeDtypeStruct((N, EPAD), jnp.float32),
            jax.ShapeDtypeStruct((N, EPAD), jnp.float32),
            jax.ShapeDtypeStruct((8, EPAD), jnp.float32),
            jax.ShapeDtypeStruct((CHUNK, EPAD), jnp.int32),
            jax.ShapeDtypeStruct((N, EPAD), jnp.float32),
            jax.ShapeDtypeStruct((N, EPAD), jnp.float32),
        ],
        scratch_shapes=[pltpu.VMEM((N, EPAD), jnp.float32)],
        compiler_params=pltpu.CompilerParams(
            vmem_limit_bytes=63 * 1024 * 1024),
    )(x2, Wp, bp)

    pos0 = sel[:, 4].astype(jnp.int32)
    pos1 = sel[:, 5].astype(jnp.int32)
    te = tmap[:N_GRID, 0]

    xs = _dispatch(x2, pos0, pos1)

    ys = pl.pallas_call(
        _gmm_body,
        grid_spec=pltpu.PrefetchScalarGridSpec(
            num_scalar_prefetch=1,
            grid=(N_GRID,),
            in_specs=[
                pl.BlockSpec((TILE, D), lambda g, te: (g, 0)),
                pl.BlockSpec((1, D, HIDDEN),
                             lambda g, te: (jnp.maximum(te[g], 0), 0, 0)),
                pl.BlockSpec((1, 1, HIDDEN),
                             lambda g, te: (jnp.maximum(te[g], 0), 0, 0)),
                pl.BlockSpec((1, HIDDEN, D),
                             lambda g, te: (jnp.maximum(te[g], 0), 0, 0)),
                pl.BlockSpec((1, 1, D),
                             lambda g, te: (jnp.maximum(te[g], 0), 0, 0)),
            ],
            out_specs=pl.BlockSpec((TILE, D), lambda g, te: (g, 0)),
        ),
        out_shape=jax.ShapeDtypeStruct((NPAD, D), jnp.float32),
        compiler_params=pltpu.CompilerParams(
            dimension_semantics=("arbitrary",),
            vmem_limit_bytes=63 * 1024 * 1024),
    )(te, xs, up_W, up_b.reshape(NUM_EXPERTS, 1, HIDDEN),
      down_W, down_b.reshape(NUM_EXPERTS, 1, D))

    out2 = _combine(ys, pos0, pos1, w0b, w1b)

    router_probs = probs_p[:, :NUM_EXPERTS].reshape(B, S, NUM_EXPERTS)
    aux_loss = misc[2, 0]
    return (out2.reshape(B, S, D), aux_loss, router_probs)


# speedup vs baseline: 1.3103x; 1.0734x over previous
import functools

import jax
import jax.numpy as jnp
from jax import lax
from jax.experimental import pallas as pl
from jax.experimental.pallas import tpu as pltpu
from jax.experimental.pallas import tpu_sc as plsc

NUM_EXPERTS = 8
TOP_K = 2
HIDDEN = 2048
D_MODEL = 1024
AUX_W = 0.01
EPAD = 128
TILE = 256
N_TOK = 4096
N_GRID = (N_TOK * TOP_K) // TILE + NUM_EXPERTS
NPAD = N_GRID * TILE
CHUNK = 128

_sc_kernels_cache = {}


def _router_body(x_ref, w_ref, b_ref,
                 probs_ref, sel_ref, misc_ref, tmap_ref, wb_ref,
                 m0_s, m1_s):
    n = x_ref.shape[0]
    nch = n // CHUNK
    x = x_ref[...]
    W = w_ref[...]
    logits = jnp.dot(x, W, preferred_element_type=jnp.float32) + b_ref[...]
    colf = lax.broadcasted_iota(jnp.int32, (n, EPAD), 1).astype(jnp.float32)
    neg = jnp.where(colf < NUM_EXPERTS, logits, -1e30)
    m0 = jnp.max(neg, axis=1, keepdims=True)
    i0 = jnp.min(jnp.where(neg == m0, colf, 1e9), axis=1, keepdims=True)
    mask0 = (colf == i0).astype(jnp.float32)
    neg1 = jnp.where(colf == i0, -1e30, neg)
    m1 = jnp.max(neg1, axis=1, keepdims=True)
    i1 = jnp.min(jnp.where(neg1 == m1, colf, 1e9), axis=1, keepdims=True)
    mask1 = (colf == i1).astype(jnp.float32)
    ex = jnp.exp(neg - m0)
    probs = ex / jnp.sum(ex, axis=1, keepdims=True)
    probs_ref[...] = probs
    t = jnp.exp(m1 - m0)
    w0 = 1.0 / (1.0 + t)
    w1 = t / (1.0 + t)
    wb_ref[...] = jnp.where(colf < 16, w0, 0.0) + jnp.where(
        (colf >= 16) & (colf < 32), w1, 0.0)
    sel = jnp.where(colf == 0, w0, 0.0)
    sel = sel + jnp.where(colf == 1, w1, 0.0)
    sel = sel + jnp.where(colf == 2, i0, 0.0)
    sel = sel + jnp.where(colf == 3, i1, 0.0)
    sel_ref[...] = sel
    m0_s[...] = mask0
    m1_s[...] = mask1
    cnt0 = jnp.sum(mask0, axis=0, keepdims=True)
    cnt1 = jnp.sum(mask1, axis=0, keepdims=True)
    psum = jnp.sum(probs, axis=0, keepdims=True)

    cnt = cnt0 + cnt1
    pm = psum / n
    aux = jnp.sum(cnt * pm) * (NUM_EXPERTS * AUX_W / (n * TOP_K))
    pc = jnp.floor((cnt + (TILE - 1)) / TILE) * TILE
    cc = lax.broadcasted_iota(jnp.int32, (CHUNK, EPAD), 1).astype(jnp.float32)
    rr = lax.broadcasted_iota(jnp.int32, (CHUNK, EPAD), 0).astype(jnp.float32)
    ltri = (cc < rr).astype(jnp.float32)
    utri = (rr < cc).astype(jnp.float32)
    padbase = jnp.dot(pc, utri, preferred_element_type=jnp.float32)

    row8 = lax.broadcasted_iota(jnp.int32, (8, EPAD), 0)
    misc = jnp.where(row8 == 0, jnp.broadcast_to(cnt, (8, EPAD)), 0.0)
    misc = misc + jnp.where(row8 == 1, jnp.broadcast_to(pm, (8, EPAD)), 0.0)
    misc = misc + jnp.where(row8 == 2, aux, 0.0)
    misc_ref[...] = misc

    pbb = padbase
    c0b = cnt0

    def posloop(c, carry):
        off0, off1 = carry
        m0c = m0_s[pl.ds(c * CHUNK, CHUNK), :]
        m1c = m1_s[pl.ds(c * CHUNK, CHUNK), :]
        cum0 = jnp.dot(ltri, m0c, preferred_element_type=jnp.float32)
        cum1 = jnp.dot(ltri, m1c, preferred_element_type=jnp.float32)
        pos0 = jnp.sum((cum0 + off0 + pbb) * m0c, axis=1, keepdims=True)
        pos1 = jnp.sum((cum1 + off1 + pbb + c0b) * m1c, axis=1, keepdims=True)
        sel_ref[pl.ds(c * CHUNK, CHUNK), :] += (
            jnp.where(cc == 4, pos0, 0.0) + jnp.where(cc == 5, pos1, 0.0))
        return (off0 + jnp.sum(m0c, axis=0, keepdims=True),
                off1 + jnp.sum(m1c, axis=0, keepdims=True))

    zero = jnp.zeros((1, EPAD), jnp.float32)
    lax.fori_loop(0, nch, posloop, (zero, zero))

    ende = jnp.broadcast_to(padbase + pc, (CHUNK, EPAD))
    gi = lax.broadcasted_iota(jnp.int32, (CHUNK, EPAD), 0).astype(jnp.float32)
    ce = lax.broadcasted_iota(jnp.int32, (CHUNK, EPAD), 1).astype(jnp.float32)
    ind = ((gi * TILE >= ende) & (ce < NUM_EXPERTS)).astype(jnp.int32)
    te = jnp.sum(ind, axis=1, keepdims=True)
    te = jnp.minimum(te, NUM_EXPERTS - 1)
    totpad = jnp.sum(pc)
    gcol = lax.broadcasted_iota(jnp.int32, (CHUNK, 1), 0).astype(jnp.float32)
    te = jnp.where(gcol * TILE >= totpad, -1, te)
    tmap_ref[...] = jnp.broadcast_to(te, (CHUNK, EPAD))


def _dispatch_body(x_hbm, p0_hbm, p1_hbm, xs_hbm, xb, idx0, idx1, s0, s1):
    wid = lax.axis_index("s") * 2 + lax.axis_index("c")
    base = wid * 128
    for sub in range(2):
        tok = pl.multiple_of(base + sub * 64, 64)
        pltpu.sync_copy(p0_hbm.at[pl.ds(tok, 64)], idx0)
        pltpu.sync_copy(p1_hbm.at[pl.ds(tok, 64)], idx1)
        pltpu.sync_copy(x_hbm.at[pl.ds(tok, 64)], xb)
        c0 = pltpu.async_copy(xb, xs_hbm.at[idx0], s0)
        c1 = pltpu.async_copy(xb, xs_hbm.at[idx1], s1)
        c0.wait()
        c1.wait()


def _gmm_body(te_ref, xs_ref, upw_ref, upb_ref, dnw_ref, dnb_ref, out_ref):
    g = pl.program_id(0)

    @pl.when(te_ref[g] >= 0)
    def _():
        x = xs_ref[...]
        h = jnp.dot(x, upw_ref[0], preferred_element_type=jnp.float32)
        h = jax.nn.gelu(h + upb_ref[0, 0][None, :])
        y = jnp.dot(h, dnw_ref[0], preferred_element_type=jnp.float32)
        out_ref[...] = y + dnb_ref[0, 0][None, :]


def _combine_body(ys_hbm, p0_hbm, p1_hbm, wb_hbm, out_hbm,
                  y0a, y1a, y0b, y1b, ob, i0a, i1a, i0b, i1b, wv,
                  s0a, s1a, s0b, s1b):
    wid = lax.axis_index("s") * 2 + lax.axis_index("c")
    base = wid * 128
    bufs = ((y0a, y1a, i0a, i1a, s0a, s1a),
            (y0b, y1b, i0b, i1b, s0b, s1b))

    def fire(ch, b):
        y0, y1, i0, i1, s0, s1 = b
        tok = pl.multiple_of(base + ch * 16, 16)
        pltpu.sync_copy(p0_hbm.at[pl.ds(tok, 16)], i0)
        pltpu.sync_copy(p1_hbm.at[pl.ds(tok, 16)], i1)
        return (pltpu.async_copy(ys_hbm.at[i0], y0, s0),
                pltpu.async_copy(ys_hbm.at[i1], y1, s1))

    cps = fire(0, bufs[0])
    for ch in range(8):
        cur = bufs[ch % 2]
        ncps = fire(ch + 1, bufs[(ch + 1) % 2]) if ch < 7 else None
        tok = pl.multiple_of(base + ch * 16, 16)
        pltpu.sync_copy(wb_hbm.at[pl.ds(tok, 16)], wv)
        cps[0].wait()
        cps[1].wait()
        y0, y1 = cur[0], cur[1]

        def row(r, _):
            w0s = wv[r, pl.ds(0, 16)]
            w1s = wv[r, pl.ds(16, 16)]
            for v in range(D_MODEL // 16):
                ob[r, pl.ds(v * 16, 16)] = (
                    w0s * y0[r, pl.ds(v * 16, 16)]
                    + w1s * y1[r, pl.ds(v * 16, 16)])
            return 0

        lax.fori_loop(0, 16, row, 0)
        pltpu.sync_copy(ob, out_hbm.at[pl.ds(tok, 16)])
        cps = ncps


def _get_sc_kernels():
    if "k" not in _sc_kernels_cache:
        mesh = plsc.VectorSubcoreMesh(core_axis_name="c", subcore_axis_name="s")
        dispatch = pl.kernel(
            _dispatch_body, mesh=mesh,
            out_type=jax.ShapeDtypeStruct((NPAD, D_MODEL), jnp.float32),
            scratch_types=[
                pltpu.VMEM((64, D_MODEL), jnp.float32),
                pltpu.VMEM((64,), jnp.int32),
                pltpu.VMEM((64,), jnp.int32),
                pltpu.SemaphoreType.DMA,
                pltpu.SemaphoreType.DMA,
            ])
        combine = pl.kernel(
            _combine_body, mesh=mesh,
            out_type=jax.ShapeDtypeStruct((N_TOK, D_MODEL), jnp.float32),
            scratch_types=[
                pltpu.VMEM((16, D_MODEL), jnp.float32),
                pltpu.VMEM((16, D_MODEL), jnp.float32),
                pltpu.VMEM((16, D_MODEL), jnp.float32),
                pltpu.VMEM((16, D_MODEL), jnp.float32),
                pltpu.VMEM((16, D_MODEL), jnp.float32),
                pltpu.VMEM((16,), jnp.int32),
                pltpu.VMEM((16,), jnp.int32),
                pltpu.VMEM((16,), jnp.int32),
                pltpu.VMEM((16,), jnp.int32),
                pltpu.VMEM((16, EPAD), jnp.float32),
                pltpu.SemaphoreType.DMA,
                pltpu.SemaphoreType.DMA,
                pltpu.SemaphoreType.DMA,
                pltpu.SemaphoreType.DMA,
            ])
        _sc_kernels_cache["k"] = (dispatch, combine)
    return _sc_kernels_cache["k"]


def kernel(x, router_W, router_b, up_W, up_b, down_W, down_b):
    _dispatch, _combine = _get_sc_kernels()
    B, S, D = x.shape
    N = B * S
    x2 = x.reshape(N, D)
    Wp = jnp.zeros((D, EPAD), jnp.float32).at[:, :NUM_EXPERTS].set(router_W)
    bp = jnp.zeros((1, EPAD), jnp.float32).at[0, :NUM_EXPERTS].set(router_b)

    probs_p, sel, misc, tmap, wb = pl.pallas_call(
        _router_body,
        out_shape=[
            jax.ShapeDtypeStruct((N, EPAD), jnp.float32),
            jax.ShapeDtypeStruct((N, EPAD), jnp.float32),
            jax.ShapeDtypeStruct((8, EPAD), jnp.float32),
            jax.ShapeDtypeStruct((CHUNK, EPAD), jnp.int32),
            jax.ShapeDtypeStruct((N, EPAD), jnp.float32),
        ],
        scratch_shapes=[pltpu.VMEM((N, EPAD), jnp.float32),
                        pltpu.VMEM((N, EPAD), jnp.float32)],
        compiler_params=pltpu.CompilerParams(
            vmem_limit_bytes=63 * 1024 * 1024),
    )(x2, Wp, bp)

    pos0 = sel[:, 4].astype(jnp.int32)
    pos1 = sel[:, 5].astype(jnp.int32)
    te = tmap[:N_GRID, 0]

    xs = _dispatch(x2, pos0, pos1)

    ys = pl.pallas_call(
        _gmm_body,
        grid_spec=pltpu.PrefetchScalarGridSpec(
            num_scalar_prefetch=1,
            grid=(N_GRID,),
            in_specs=[
                pl.BlockSpec((TILE, D), lambda g, te: (g, 0)),
                pl.BlockSpec((1, D, HIDDEN),
                             lambda g, te: (jnp.maximum(te[g], 0), 0, 0)),
                pl.BlockSpec((1, 1, HIDDEN),
                             lambda g, te: (jnp.maximum(te[g], 0), 0, 0)),
                pl.BlockSpec((1, HIDDEN, D),
                             lambda g, te: (jnp.maximum(te[g], 0), 0, 0)),
                pl.BlockSpec((1, 1, D),
                             lambda g, te: (jnp.maximum(te[g], 0), 0, 0)),
            ],
            out_specs=pl.BlockSpec((TILE, D), lambda g, te: (g, 0)),
        ),
        out_shape=jax.ShapeDtypeStruct((NPAD, D), jnp.float32),
        compiler_params=pltpu.CompilerParams(
            dimension_semantics=("arbitrary",),
            vmem_limit_bytes=63 * 1024 * 1024),
    )(te, xs, up_W, up_b.reshape(NUM_EXPERTS, 1, HIDDEN),
      down_W, down_b.reshape(NUM_EXPERTS, 1, D))

    out2 = _combine(ys, pos0, pos1, wb)

    router_probs = probs_p[:, :NUM_EXPERTS].reshape(B, S, NUM_EXPERTS)
    aux_loss = misc[2, 0]
    return (out2.reshape(B, S, D), aux_loss, router_probs)


# speedup vs baseline: 1.3931x; 1.0632x over previous
import functools

import jax
import jax.numpy as jnp
from jax import lax
from jax.experimental import pallas as pl
from jax.experimental.pallas import tpu as pltpu
from jax.experimental.pallas import tpu_sc as plsc

NUM_EXPERTS = 8
TOP_K = 2
HIDDEN = 2048
D_MODEL = 1024
AUX_W = 0.01
EPAD = 128
TILE = 512
N_TOK = 4096
N_GRID = (N_TOK * TOP_K) // TILE + NUM_EXPERTS
NPAD = N_GRID * TILE
CHUNK = 128

_sc_kernels_cache = {}


def _router_body(x_ref, w_ref, b_ref,
                 probs_ref, sel_ref, misc_ref, tmap_ref, wb_ref,
                 m0_s, m1_s):
    n = x_ref.shape[0]
    nch = n // CHUNK
    x = x_ref[...]
    W = w_ref[...]
    logits = jnp.dot(x, W, preferred_element_type=jnp.float32) + b_ref[...]
    colf = lax.broadcasted_iota(jnp.int32, (n, EPAD), 1).astype(jnp.float32)
    neg = jnp.where(colf < NUM_EXPERTS, logits, -1e30)
    m0 = jnp.max(neg, axis=1, keepdims=True)
    i0 = jnp.min(jnp.where(neg == m0, colf, 1e9), axis=1, keepdims=True)
    mask0 = (colf == i0).astype(jnp.float32)
    neg1 = jnp.where(colf == i0, -1e30, neg)
    m1 = jnp.max(neg1, axis=1, keepdims=True)
    i1 = jnp.min(jnp.where(neg1 == m1, colf, 1e9), axis=1, keepdims=True)
    mask1 = (colf == i1).astype(jnp.float32)
    ex = jnp.exp(neg - m0)
    probs = ex / jnp.sum(ex, axis=1, keepdims=True)
    probs_ref[...] = probs
    t = jnp.exp(m1 - m0)
    w0 = 1.0 / (1.0 + t)
    w1 = t / (1.0 + t)
    wb_ref[...] = jnp.where(colf < 16, w0, 0.0) + jnp.where(
        (colf >= 16) & (colf < 32), w1, 0.0)
    sel = jnp.where(colf == 0, w0, 0.0)
    sel = sel + jnp.where(colf == 1, w1, 0.0)
    sel = sel + jnp.where(colf == 2, i0, 0.0)
    sel = sel + jnp.where(colf == 3, i1, 0.0)
    sel_ref[...] = sel
    m0_s[...] = mask0
    m1_s[...] = mask1
    cnt0 = jnp.sum(mask0, axis=0, keepdims=True)
    cnt1 = jnp.sum(mask1, axis=0, keepdims=True)
    psum = jnp.sum(probs, axis=0, keepdims=True)

    cnt = cnt0 + cnt1
    pm = psum / n
    aux = jnp.sum(cnt * pm) * (NUM_EXPERTS * AUX_W / (n * TOP_K))
    pc = jnp.floor((cnt + (TILE - 1)) / TILE) * TILE
    cc = lax.broadcasted_iota(jnp.int32, (CHUNK, EPAD), 1).astype(jnp.float32)
    rr = lax.broadcasted_iota(jnp.int32, (CHUNK, EPAD), 0).astype(jnp.float32)
    ltri = (cc < rr).astype(jnp.float32)
    utri = (rr < cc).astype(jnp.float32)
    padbase = jnp.dot(pc, utri, preferred_element_type=jnp.float32)

    row8 = lax.broadcasted_iota(jnp.int32, (8, EPAD), 0)
    misc = jnp.where(row8 == 0, jnp.broadcast_to(cnt, (8, EPAD)), 0.0)
    misc = misc + jnp.where(row8 == 1, jnp.broadcast_to(pm, (8, EPAD)), 0.0)
    misc = misc + jnp.where(row8 == 2, aux, 0.0)
    misc_ref[...] = misc

    pbb = padbase
    c0b = cnt0

    def posloop(c, carry):
        off0, off1 = carry
        m0c = m0_s[pl.ds(c * CHUNK, CHUNK), :]
        m1c = m1_s[pl.ds(c * CHUNK, CHUNK), :]
        cum0 = jnp.dot(ltri, m0c, preferred_element_type=jnp.float32)
        cum1 = jnp.dot(ltri, m1c, preferred_element_type=jnp.float32)
        pos0 = jnp.sum((cum0 + off0 + pbb) * m0c, axis=1, keepdims=True)
        pos1 = jnp.sum((cum1 + off1 + pbb + c0b) * m1c, axis=1, keepdims=True)
        sel_ref[pl.ds(c * CHUNK, CHUNK), :] += (
            jnp.where(cc == 4, pos0, 0.0) + jnp.where(cc == 5, pos1, 0.0))
        return (off0 + jnp.sum(m0c, axis=0, keepdims=True),
                off1 + jnp.sum(m1c, axis=0, keepdims=True))

    zero = jnp.zeros((1, EPAD), jnp.float32)
    lax.fori_loop(0, nch, posloop, (zero, zero))

    ende = jnp.broadcast_to(padbase + pc, (CHUNK, EPAD))
    gi = lax.broadcasted_iota(jnp.int32, (CHUNK, EPAD), 0).astype(jnp.float32)
    ce = lax.broadcasted_iota(jnp.int32, (CHUNK, EPAD), 1).astype(jnp.float32)
    ind = ((gi * TILE >= ende) & (ce < NUM_EXPERTS)).astype(jnp.int32)
    te = jnp.sum(ind, axis=1, keepdims=True)
    te = jnp.minimum(te, NUM_EXPERTS - 1)
    totpad = jnp.sum(pc)
    gcol = lax.broadcasted_iota(jnp.int32, (CHUNK, 1), 0).astype(jnp.float32)
    te = jnp.where(gcol * TILE >= totpad, -1, te)
    tmap_ref[...] = jnp.broadcast_to(te, (CHUNK, EPAD))


def _dispatch_body(x_hbm, p0_hbm, p1_hbm, xs_hbm, xb, idx0, idx1, s0, s1):
    wid = lax.axis_index("s") * 2 + lax.axis_index("c")
    base = wid * 128
    for sub in range(2):
        tok = pl.multiple_of(base + sub * 64, 64)
        pltpu.sync_copy(p0_hbm.at[pl.ds(tok, 64)], idx0)
        pltpu.sync_copy(p1_hbm.at[pl.ds(tok, 64)], idx1)
        pltpu.sync_copy(x_hbm.at[pl.ds(tok, 64)], xb)
        c0 = pltpu.async_copy(xb, xs_hbm.at[idx0], s0)
        c1 = pltpu.async_copy(xb, xs_hbm.at[idx1], s1)
        c0.wait()
        c1.wait()


def _gmm_body(te_ref, xs_ref, upw_ref, upb_ref, dnw_ref, dnb_ref, out_ref):
    g = pl.program_id(0)

    @pl.when(te_ref[g] >= 0)
    def _():
        x = xs_ref[...]
        h = jnp.dot(x, upw_ref[0], preferred_element_type=jnp.float32)
        h = jax.nn.gelu(h + upb_ref[0, 0][None, :])
        y = jnp.dot(h, dnw_ref[0], preferred_element_type=jnp.float32)
        out_ref[...] = y + dnb_ref[0, 0][None, :]


def _combine_body(ys_hbm, p0_hbm, p1_hbm, wb_hbm, out_hbm,
                  y0a, y1a, y0b, y1b, ob, i0a, i1a, i0b, i1b, wv,
                  s0a, s1a, s0b, s1b):
    wid = lax.axis_index("s") * 2 + lax.axis_index("c")
    base = wid * 128
    bufs = ((y0a, y1a, i0a, i1a, s0a, s1a),
            (y0b, y1b, i0b, i1b, s0b, s1b))

    def fire(ch, b):
        y0, y1, i0, i1, s0, s1 = b
        tok = pl.multiple_of(base + ch * 16, 16)
        pltpu.sync_copy(p0_hbm.at[pl.ds(tok, 16)], i0)
        pltpu.sync_copy(p1_hbm.at[pl.ds(tok, 16)], i1)
        return (pltpu.async_copy(ys_hbm.at[i0], y0, s0),
                pltpu.async_copy(ys_hbm.at[i1], y1, s1))

    cps = fire(0, bufs[0])
    for ch in range(8):
        cur = bufs[ch % 2]
        ncps = fire(ch + 1, bufs[(ch + 1) % 2]) if ch < 7 else None
        tok = pl.multiple_of(base + ch * 16, 16)
        pltpu.sync_copy(wb_hbm.at[pl.ds(tok, 16)], wv)
        cps[0].wait()
        cps[1].wait()
        y0, y1 = cur[0], cur[1]

        def row(r, _):
            w0s = wv[r, pl.ds(0, 16)]
            w1s = wv[r, pl.ds(16, 16)]
            for v in range(D_MODEL // 16):
                ob[r, pl.ds(v * 16, 16)] = (
                    w0s * y0[r, pl.ds(v * 16, 16)]
                    + w1s * y1[r, pl.ds(v * 16, 16)])
            return 0

        lax.fori_loop(0, 16, row, 0)
        pltpu.sync_copy(ob, out_hbm.at[pl.ds(tok, 16)])
        cps = ncps


def _get_sc_kernels():
    if "k" not in _sc_kernels_cache:
        mesh = plsc.VectorSubcoreMesh(core_axis_name="c", subcore_axis_name="s")
        dispatch = pl.kernel(
            _dispatch_body, mesh=mesh,
            out_type=jax.ShapeDtypeStruct((NPAD, D_MODEL), jnp.float32),
            scratch_types=[
                pltpu.VMEM((64, D_MODEL), jnp.float32),
                pltpu.VMEM((64,), jnp.int32),
                pltpu.VMEM((64,), jnp.int32),
                pltpu.SemaphoreType.DMA,
                pltpu.SemaphoreType.DMA,
            ])
        combine = pl.kernel(
            _combine_body, mesh=mesh,
            out_type=jax.ShapeDtypeStruct((N_TOK, D_MODEL), jnp.float32),
            scratch_types=[
                pltpu.VMEM((16, D_MODEL), jnp.float32),
                pltpu.VMEM((16, D_MODEL), jnp.float32),
                pltpu.VMEM((16, D_MODEL), jnp.float32),
                pltpu.VMEM((16, D_MODEL), jnp.float32),
                pltpu.VMEM((16, D_MODEL), jnp.float32),
                pltpu.VMEM((16,), jnp.int32),
                pltpu.VMEM((16,), jnp.int32),
                pltpu.VMEM((16,), jnp.int32),
                pltpu.VMEM((16,), jnp.int32),
                pltpu.VMEM((16, EPAD), jnp.float32),
                pltpu.SemaphoreType.DMA,
                pltpu.SemaphoreType.DMA,
                pltpu.SemaphoreType.DMA,
                pltpu.SemaphoreType.DMA,
            ])
        _sc_kernels_cache["k"] = (dispatch, combine)
    return _sc_kernels_cache["k"]


def kernel(x, router_W, router_b, up_W, up_b, down_W, down_b):
    _dispatch, _combine = _get_sc_kernels()
    B, S, D = x.shape
    N = B * S
    x2 = x.reshape(N, D)
    Wp = jnp.zeros((D, EPAD), jnp.float32).at[:, :NUM_EXPERTS].set(router_W)
    bp = jnp.zeros((1, EPAD), jnp.float32).at[0, :NUM_EXPERTS].set(router_b)

    probs_p, sel, misc, tmap, wb = pl.pallas_call(
        _router_body,
        out_shape=[
            jax.ShapeDtypeStruct((N, EPAD), jnp.float32),
            jax.ShapeDtypeStruct((N, EPAD), jnp.float32),
            jax.ShapeDtypeStruct((8, EPAD), jnp.float32),
            jax.ShapeDtypeStruct((CHUNK, EPAD), jnp.int32),
            jax.ShapeDtypeStruct((N, EPAD), jnp.float32),
        ],
        scratch_shapes=[pltpu.VMEM((N, EPAD), jnp.float32),
                        pltpu.VMEM((N, EPAD), jnp.float32)],
        compiler_params=pltpu.CompilerParams(
            vmem_limit_bytes=63 * 1024 * 1024),
    )(x2, Wp, bp)

    pos0 = sel[:, 4].astype(jnp.int32)
    pos1 = sel[:, 5].astype(jnp.int32)
    te = tmap[:N_GRID, 0]

    xs = _dispatch(x2, pos0, pos1)

    ys = pl.pallas_call(
        _gmm_body,
        grid_spec=pltpu.PrefetchScalarGridSpec(
            num_scalar_prefetch=1,
            grid=(N_GRID,),
            in_specs=[
                pl.BlockSpec((TILE, D), lambda g, te: (g, 0)),
                pl.BlockSpec((1, D, HIDDEN),
                             lambda g, te: (jnp.maximum(te[g], 0), 0, 0)),
                pl.BlockSpec((1, 1, HIDDEN),
                             lambda g, te: (jnp.maximum(te[g], 0), 0, 0)),
                pl.BlockSpec((1, HIDDEN, D),
                             lambda g, te: (jnp.maximum(te[g], 0), 0, 0)),
                pl.BlockSpec((1, 1, D),
                             lambda g, te: (jnp.maximum(te[g], 0), 0, 0)),
            ],
            out_specs=pl.BlockSpec((TILE, D), lambda g, te: (g, 0)),
        ),
        out_shape=jax.ShapeDtypeStruct((NPAD, D), jnp.float32),
        compiler_params=pltpu.CompilerParams(
            dimension_semantics=("arbitrary",),
            vmem_limit_bytes=63 * 1024 * 1024),
    )(te, xs, up_W, up_b.reshape(NUM_EXPERTS, 1, HIDDEN),
      down_W, down_b.reshape(NUM_EXPERTS, 1, D))

    out2 = _combine(ys, pos0, pos1, wb)

    router_probs = probs_p[:, :NUM_EXPERTS].reshape(B, S, NUM_EXPERTS)
    aux_loss = misc[2, 0]
    return (out2.reshape(B, S, D), aux_loss, router_probs)


# speedup vs baseline: 1.4869x; 1.0673x over previous
import functools

import jax
import jax.numpy as jnp
from jax import lax
from jax.experimental import pallas as pl
from jax.experimental.pallas import tpu as pltpu
from jax.experimental.pallas import tpu_sc as plsc

NUM_EXPERTS = 8
TOP_K = 2
HIDDEN = 2048
D_MODEL = 1024
AUX_W = 0.01
EPAD = 128
TILE = 512
N_TOK = 4096
N_GRID = (N_TOK * TOP_K) // TILE + NUM_EXPERTS
NPAD = N_GRID * TILE
CHUNK = 128

_sc_kernels_cache = {}


def _router_body(x_ref, w_ref, b_ref,
                 probs_ref, sel_ref, misc_ref, tmap_ref, wb_ref,
                 m0_s, m1_s):
    n = x_ref.shape[0]
    nch = n // CHUNK
    x = x_ref[...]
    W = w_ref[...]
    logits = jnp.dot(x, W, preferred_element_type=jnp.float32) + b_ref[...]
    colf = lax.broadcasted_iota(jnp.int32, (n, EPAD), 1).astype(jnp.float32)
    neg = jnp.where(colf < NUM_EXPERTS, logits, -1e30)
    m0 = jnp.max(neg, axis=1, keepdims=True)
    i0 = jnp.min(jnp.where(neg == m0, colf, 1e9), axis=1, keepdims=True)
    mask0 = (colf == i0).astype(jnp.float32)
    neg1 = jnp.where(colf == i0, -1e30, neg)
    m1 = jnp.max(neg1, axis=1, keepdims=True)
    i1 = jnp.min(jnp.where(neg1 == m1, colf, 1e9), axis=1, keepdims=True)
    mask1 = (colf == i1).astype(jnp.float32)
    ex = jnp.exp(neg - m0)
    probs = ex / jnp.sum(ex, axis=1, keepdims=True)
    probs_ref[...] = probs
    t = jnp.exp(m1 - m0)
    w0 = 1.0 / (1.0 + t)
    w1 = t / (1.0 + t)
    wb_ref[...] = jnp.where(colf < 16, w0, 0.0) + jnp.where(
        (colf >= 16) & (colf < 32), w1, 0.0)
    sel = jnp.where(colf == 0, w0, 0.0)
    sel = sel + jnp.where(colf == 1, w1, 0.0)
    sel = sel + jnp.where(colf == 2, i0, 0.0)
    sel = sel + jnp.where(colf == 3, i1, 0.0)
    sel_ref[...] = sel
    m0_s[...] = mask0
    m1_s[...] = mask1
    cnt0 = jnp.sum(mask0, axis=0, keepdims=True)
    cnt1 = jnp.sum(mask1, axis=0, keepdims=True)
    psum = jnp.sum(probs, axis=0, keepdims=True)

    cnt = cnt0 + cnt1
    pm = psum / n
    aux = jnp.sum(cnt * pm) * (NUM_EXPERTS * AUX_W / (n * TOP_K))
    pc = jnp.floor((cnt + (TILE - 1)) / TILE) * TILE
    cc = lax.broadcasted_iota(jnp.int32, (CHUNK, EPAD), 1).astype(jnp.float32)
    rr = lax.broadcasted_iota(jnp.int32, (CHUNK, EPAD), 0).astype(jnp.float32)
    ltri = (cc < rr).astype(jnp.float32)
    utri = (rr < cc).astype(jnp.float32)
    padbase = jnp.dot(pc, utri, preferred_element_type=jnp.float32)

    row8 = lax.broadcasted_iota(jnp.int32, (8, EPAD), 0)
    misc = jnp.where(row8 == 0, jnp.broadcast_to(cnt, (8, EPAD)), 0.0)
    misc = misc + jnp.where(row8 == 1, jnp.broadcast_to(pm, (8, EPAD)), 0.0)
    misc = misc + jnp.where(row8 == 2, aux, 0.0)
    misc_ref[...] = misc

    pbb = padbase
    c0b = cnt0

    def posloop(c, carry):
        off0, off1 = carry
        m0c = m0_s[pl.ds(c * CHUNK, CHUNK), :]
        m1c = m1_s[pl.ds(c * CHUNK, CHUNK), :]
        cum0 = jnp.dot(ltri, m0c, preferred_element_type=jnp.float32)
        cum1 = jnp.dot(ltri, m1c, preferred_element_type=jnp.float32)
        pos0 = jnp.sum((cum0 + off0 + pbb) * m0c, axis=1, keepdims=True)
        pos1 = jnp.sum((cum1 + off1 + pbb + c0b) * m1c, axis=1, keepdims=True)
        sel_ref[pl.ds(c * CHUNK, CHUNK), :] += (
            jnp.where(cc == 4, pos0, 0.0) + jnp.where(cc == 5, pos1, 0.0))
        return (off0 + jnp.sum(m0c, axis=0, keepdims=True),
                off1 + jnp.sum(m1c, axis=0, keepdims=True))

    zero = jnp.zeros((1, EPAD), jnp.float32)
    lax.fori_loop(0, nch, posloop, (zero, zero))

    ende = jnp.broadcast_to(padbase + pc, (CHUNK, EPAD))
    gi = lax.broadcasted_iota(jnp.int32, (CHUNK, EPAD), 0).astype(jnp.float32)
    ce = lax.broadcasted_iota(jnp.int32, (CHUNK, EPAD), 1).astype(jnp.float32)
    ind = ((gi * TILE >= ende) & (ce < NUM_EXPERTS)).astype(jnp.int32)
    te = jnp.sum(ind, axis=1, keepdims=True)
    te = jnp.minimum(te, NUM_EXPERTS - 1)
    totpad = jnp.sum(pc)
    gcol = lax.broadcasted_iota(jnp.int32, (CHUNK, 1), 0).astype(jnp.float32)
    te = jnp.where(gcol * TILE >= totpad, -1, te)
    tmap_ref[...] = jnp.broadcast_to(te, (CHUNK, EPAD))


def _dispatch_body(x_hbm, p0_hbm, p1_hbm, xs_hbm, xb, idx0, idx1, s0, s1):
    wid = lax.axis_index("s") * 2 + lax.axis_index("c")
    base = wid * 128
    for sub in range(2):
        tok = pl.multiple_of(base + sub * 64, 64)
        pltpu.sync_copy(p0_hbm.at[pl.ds(tok, 64)], idx0)
        pltpu.sync_copy(p1_hbm.at[pl.ds(tok, 64)], idx1)
        pltpu.sync_copy(x_hbm.at[pl.ds(tok, 64)], xb)
        c0 = pltpu.async_copy(xb, xs_hbm.at[idx0], s0)
        c1 = pltpu.async_copy(xb, xs_hbm.at[idx1], s1)
        c0.wait()
        c1.wait()


def _gmm_body(te_ref, xs_ref, upw_ref, upb_ref, dnw_ref, dnb_ref, out_ref):
    g = pl.program_id(0)

    @pl.when(te_ref[g] >= 0)
    def _():
        x = xs_ref[...]
        h = jnp.dot(x, upw_ref[0], preferred_element_type=jnp.float32)
        h = jax.nn.gelu(h + upb_ref[0, 0][None, :])
        y = jnp.dot(h, dnw_ref[0], preferred_element_type=jnp.float32)
        out_ref[...] = y + dnb_ref[0, 0][None, :]


def _combine_body(ys_hbm, p0_hbm, p1_hbm, wb_hbm, out_hbm,
                  y0a, y1a, y0b, y1b, oba, obb, idx0, idx1, wv,
                  s0a, s1a, s0b, s1b, so):
    wid = lax.axis_index("s") * 2 + lax.axis_index("c")
    base = pl.multiple_of(wid * 128, 128)
    pltpu.sync_copy(p0_hbm.at[pl.ds(base, 128)], idx0)
    pltpu.sync_copy(p1_hbm.at[pl.ds(base, 128)], idx1)
    pltpu.sync_copy(wb_hbm.at[pl.ds(base, 128)], wv)
    bufs = ((y0a, y1a, s0a, s1a, oba), (y0b, y1b, s0b, s1b, obb))

    def fire(ch, b):
        y0, y1, s0, s1, _ = b
        return (pltpu.async_copy(ys_hbm.at[idx0.at[pl.ds(ch * 16, 16)]], y0, s0),
                pltpu.async_copy(ys_hbm.at[idx1.at[pl.ds(ch * 16, 16)]], y1, s1))

    cps = fire(0, bufs[0])
    ocp = (None, None)
    for ch in range(8):
        cur = bufs[ch % 2]
        ncps = fire(ch + 1, bufs[(ch + 1) % 2]) if ch < 7 else None
        cps[0].wait()
        cps[1].wait()
        y0, y1, _, _, ob = cur
        if ocp[ch % 2] is not None:
            ocp[ch % 2].wait()

        def row(r, _):
            rr = ch * 16 + r
            w0s = wv[rr, pl.ds(0, 16)]
            w1s = wv[rr, pl.ds(16, 16)]
            for v in range(D_MODEL // 16):
                ob[r, pl.ds(v * 16, 16)] = (
                    w0s * y0[r, pl.ds(v * 16, 16)]
                    + w1s * y1[r, pl.ds(v * 16, 16)])
            return 0

        lax.fori_loop(0, 16, row, 0)
        tok = pl.multiple_of(base + ch * 16, 16)
        oc = pltpu.async_copy(ob, out_hbm.at[pl.ds(tok, 16)], so)
        ocp = (oc, ocp[1]) if ch % 2 == 0 else (ocp[0], oc)
        cps = ncps
    for c in ocp:
        if c is not None:
            c.wait()


def _get_sc_kernels():
    if "k" not in _sc_kernels_cache:
        mesh = plsc.VectorSubcoreMesh(core_axis_name="c", subcore_axis_name="s")
        dispatch = pl.kernel(
            _dispatch_body, mesh=mesh,
            out_type=jax.ShapeDtypeStruct((NPAD, D_MODEL), jnp.float32),
            scratch_types=[
                pltpu.VMEM((64, D_MODEL), jnp.float32),
                pltpu.VMEM((64,), jnp.int32),
                pltpu.VMEM((64,), jnp.int32),
                pltpu.SemaphoreType.DMA,
                pltpu.SemaphoreType.DMA,
            ])
        combine = pl.kernel(
            _combine_body, mesh=mesh,
            out_type=jax.ShapeDtypeStruct((N_TOK, D_MODEL), jnp.float32),
            scratch_types=[
                pltpu.VMEM((16, D_MODEL), jnp.float32),
                pltpu.VMEM((16, D_MODEL), jnp.float32),
                pltpu.VMEM((16, D_MODEL), jnp.float32),
                pltpu.VMEM((16, D_MODEL), jnp.float32),
                pltpu.VMEM((16, D_MODEL), jnp.float32),
                pltpu.VMEM((16, D_MODEL), jnp.float32),
                pltpu.VMEM((128,), jnp.int32),
                pltpu.VMEM((128,), jnp.int32),
                pltpu.VMEM((128, EPAD), jnp.float32),
                pltpu.SemaphoreType.DMA,
                pltpu.SemaphoreType.DMA,
                pltpu.SemaphoreType.DMA,
                pltpu.SemaphoreType.DMA,
                pltpu.SemaphoreType.DMA,
            ])
        _sc_kernels_cache["k"] = (dispatch, combine)
    return _sc_kernels_cache["k"]


def kernel(x, router_W, router_b, up_W, up_b, down_W, down_b):
    _dispatch, _combine = _get_sc_kernels()
    B, S, D = x.shape
    N = B * S
    x2 = x.reshape(N, D)
    Wp = jnp.zeros((D, EPAD), jnp.float32).at[:, :NUM_EXPERTS].set(router_W)
    bp = jnp.zeros((1, EPAD), jnp.float32).at[0, :NUM_EXPERTS].set(router_b)

    probs_p, sel, misc, tmap, wb = pl.pallas_call(
        _router_body,
        out_shape=[
            jax.ShapeDtypeStruct((N, EPAD), jnp.float32),
            jax.ShapeDtypeStruct((N, EPAD), jnp.float32),
            jax.ShapeDtypeStruct((8, EPAD), jnp.float32),
            jax.ShapeDtypeStruct((CHUNK, EPAD), jnp.int32),
            jax.ShapeDtypeStruct((N, EPAD), jnp.float32),
        ],
        scratch_shapes=[pltpu.VMEM((N, EPAD), jnp.float32),
                        pltpu.VMEM((N, EPAD), jnp.float32)],
        compiler_params=pltpu.CompilerParams(
            vmem_limit_bytes=63 * 1024 * 1024),
    )(x2, Wp, bp)

    pos0 = sel[:, 4].astype(jnp.int32)
    pos1 = sel[:, 5].astype(jnp.int32)
    te = tmap[:N_GRID, 0]

    xs = _dispatch(x2, pos0, pos1)

    ys = pl.pallas_call(
        _gmm_body,
        grid_spec=pltpu.PrefetchScalarGridSpec(
            num_scalar_prefetch=1,
            grid=(N_GRID,),
            in_specs=[
                pl.BlockSpec((TILE, D), lambda g, te: (g, 0)),
                pl.BlockSpec((1, D, HIDDEN),
                             lambda g, te: (jnp.maximum(te[g], 0), 0, 0)),
                pl.BlockSpec((1, 1, HIDDEN),
                             lambda g, te: (jnp.maximum(te[g], 0), 0, 0)),
                pl.BlockSpec((1, HIDDEN, D),
                             lambda g, te: (jnp.maximum(te[g], 0), 0, 0)),
                pl.BlockSpec((1, 1, D),
                             lambda g, te: (jnp.maximum(te[g], 0), 0, 0)),
            ],
            out_specs=pl.BlockSpec((TILE, D), lambda g, te: (g, 0)),
        ),
        out_shape=jax.ShapeDtypeStruct((NPAD, D), jnp.float32),
        compiler_params=pltpu.CompilerParams(
            dimension_semantics=("arbitrary",),
            vmem_limit_bytes=63 * 1024 * 1024),
    )(te, xs, up_W, up_b.reshape(NUM_EXPERTS, 1, HIDDEN),
      down_W, down_b.reshape(NUM_EXPERTS, 1, D))

    out2 = _combine(ys, pos0, pos1, wb)

    router_probs = probs_p[:, :NUM_EXPERTS].reshape(B, S, NUM_EXPERTS)
    aux_loss = misc[2, 0]
    return (out2.reshape(B, S, D), aux_loss, router_probs)
